# Initial kernel scaffold; baseline (speedup 1.0000x reference)
import jax
import jax.numpy as jnp
from jax.experimental import pallas as pl


def kernel(x, gate, e_bias, w1, w2, su, sd):
    raise NotImplementedError("write your pallas kernel here")


# speedup vs baseline: 1.8784x; 1.8784x over previous
import functools

import jax
import jax.numpy as jnp
from jax import lax
from jax.experimental import pallas as pl
from jax.experimental.pallas import tpu as pltpu

E = 64
TOP_K = 2
N_GROUP = 8
GSZ = E // N_GROUP
TOPK_GROUP = 4
DIM = 1024
HID = 512
T = 2048
S = T * TOP_K
B_TILE = 128
NT_PAD = S // B_TILE + E - 1
SROWS = NT_PAD * B_TILE
DUMP = 128
NEG = -1e30


def _relu2(v):
    return jnp.square(jnp.maximum(v, 0.0))


def _routing_body(x_ref, gate_ref, bias_ref, sel_ref, prob_ref):
    bt = x_ref.shape[0]
    s = jax.nn.sigmoid(
        lax.dot_general(x_ref[...], gate_ref[...], (((1,), (1,)), ((), ())),
                        preferred_element_type=jnp.float32))
    sfc = s + bias_ref[...]

    col8 = lax.broadcasted_iota(jnp.int32, (bt, GSZ), 1)
    gparts = []
    for g in range(N_GROUP):
        sg = sfc[:, g * GSZ:(g + 1) * GSZ]
        m1 = jnp.max(sg, axis=1, keepdims=True)
        i1 = jnp.min(jnp.where(sg == m1, col8, GSZ), axis=1, keepdims=True)
        m2 = jnp.max(jnp.where(col8 == i1, NEG, sg), axis=1, keepdims=True)
        gparts.append(m1 + m2)
    gscore = jnp.concatenate(gparts, axis=1)

    gmask = jnp.zeros((bt, N_GROUP), jnp.float32)
    work = gscore
    for _ in range(TOPK_GROUP):
        mx = jnp.max(work, axis=1, keepdims=True)
        ix = jnp.min(jnp.where(work == mx, col8, N_GROUP), axis=1,
                     keepdims=True)
        hit = col8 == ix
        gmask = jnp.where(hit, 1.0, gmask)
        work = jnp.where(hit, NEG, work)

    smask = jnp.concatenate(
        [jnp.broadcast_to(gmask[:, g:g + 1], (bt, GSZ)) for g in range(N_GROUP)],
        axis=1)
    sfc_m = jnp.where(smask > 0, sfc, 0.0)

    colE = lax.broadcasted_iota(jnp.int32, (bt, E), 1)
    mx1 = jnp.max(sfc_m, axis=1, keepdims=True)
    i1 = jnp.min(jnp.where(sfc_m == mx1, colE, E), axis=1, keepdims=True)
    w2m = jnp.where(colE == i1, NEG, sfc_m)
    mx2 = jnp.max(w2m, axis=1, keepdims=True)
    i2 = jnp.min(jnp.where(w2m == mx2, colE, E), axis=1, keepdims=True)

    s1 = jnp.sum(jnp.where(colE == i1, s, 0.0), axis=1, keepdims=True)
    s2 = jnp.sum(jnp.where(colE == i2, s, 0.0), axis=1, keepdims=True)
    den = s1 + s2 + 1e-20
    sel_ref[...] = jnp.concatenate([i1, i2], axis=1)
    prob_ref[...] = jnp.concatenate([s1 / den, s2 / den], axis=1)


def _routing(x, gate, e_bias):
    bt = 256
    grid = (T // bt,)
    return pl.pallas_call(
        _routing_body,
        grid=grid,
        in_specs=[
            pl.BlockSpec((bt, DIM), lambda i: (i, 0)),
            pl.BlockSpec((E, DIM), lambda i: (0, 0)),
            pl.BlockSpec((1, E), lambda i: (0, 0)),
        ],
        out_specs=[
            pl.BlockSpec((bt, TOP_K), lambda i: (i, 0)),
            pl.BlockSpec((bt, TOP_K), lambda i: (i, 0)),
        ],
        out_shape=[
            jax.ShapeDtypeStruct((T, TOP_K), jnp.int32),
            jax.ShapeDtypeStruct((T, TOP_K), jnp.float32),
        ],
    )(x, gate, e_bias.reshape(1, E))


def _dispatch_jnp(sel, probs):
    e_flat = sel.reshape(-1)
    p_flat = probs.reshape(-1)
    counts = jnp.zeros((E,), jnp.int32).at[e_flat].add(1)
    tiles = (counts + B_TILE - 1) // B_TILE
    pad_off = (jnp.cumsum(tiles) - tiles) * B_TILE
    order = jnp.argsort(e_flat, stable=True)
    e_sorted = e_flat[order]
    start = jnp.cumsum(counts) - counts
    rank = jnp.arange(S, dtype=jnp.int32) - start[e_sorted]
    pos = pad_off[e_sorted] + rank
    row_token = jnp.zeros((SROWS + DUMP,), jnp.int32).at[pos].set(
        (order // TOP_K).astype(jnp.int32))
    row_prob = jnp.zeros((SROWS + DUMP,), jnp.float32).at[pos].set(
        p_flat[order])
    return row_token, row_prob, counts


def _metadata(counts):
    tiles = (counts + B_TILE - 1) // B_TILE
    tile_cum = jnp.cumsum(tiles)
    nt_used = tile_cum[-1].astype(jnp.int32)
    s_arange = jnp.arange(NT_PAD, dtype=jnp.int32)
    step_tile = jnp.minimum(s_arange, nt_used - 1)
    step_expert = jnp.searchsorted(tile_cum, step_tile, side="right")
    step_expert = jnp.clip(step_expert, 0, E - 1).astype(jnp.int32)
    return step_tile, step_expert, nt_used.reshape(1)


def _gather_jnp(x, row_token):
    return x[row_token[:SROWS]]


def _gmm_body(step_tile, step_expert, nsteps, x_ref, w1_ref, w2_ref, p_ref,
              out_ref):
    i = pl.program_id(0)

    @pl.when(i < nsteps[0])
    def _():
        z = _relu2(lax.dot_general(
            x_ref[...], w1_ref[0], (((1,), (1,)), ((), ())),
            preferred_element_type=jnp.float32))
        y = lax.dot_general(z, w2_ref[0], (((1,), (1,)), ((), ())),
                            preferred_element_type=jnp.float32)
        out_ref[...] = y * p_ref[...]


def _gmm(x_sorted, w1, w2, row_prob, step_tile, step_expert, nsteps):
    grid_spec = pltpu.PrefetchScalarGridSpec(
        num_scalar_prefetch=3,
        grid=(NT_PAD,),
        in_specs=[
            pl.BlockSpec((B_TILE, DIM), lambda i, st, se, ns: (st[i], 0)),
            pl.BlockSpec((1, HID, DIM), lambda i, st, se, ns: (se[i], 0, 0)),
            pl.BlockSpec((1, DIM, HID), lambda i, st, se, ns: (se[i], 0, 0)),
            pl.BlockSpec((B_TILE, 1), lambda i, st, se, ns: (st[i], 0)),
        ],
        out_specs=pl.BlockSpec((B_TILE, DIM), lambda i, st, se, ns: (st[i], 0)),
    )
    return pl.pallas_call(
        _gmm_body,
        grid_spec=grid_spec,
        out_shape=jax.ShapeDtypeStruct((SROWS, DIM), jnp.float32),
    )(step_tile, step_expert, nsteps, x_sorted, w1, w2,
      row_prob[:SROWS].reshape(SROWS, 1))


def _shared_body(x_ref, su_ref, sd_ref, out_ref):
    z = _relu2(lax.dot_general(x_ref[...], su_ref[...],
                               (((1,), (1,)), ((), ())),
                               preferred_element_type=jnp.float32))
    out_ref[...] = lax.dot_general(z, sd_ref[...], (((1,), (1,)), ((), ())),
                                   preferred_element_type=jnp.float32)


def _shared(x, su, sd):
    bt = 256
    return pl.pallas_call(
        _shared_body,
        grid=(T // bt,),
        in_specs=[
            pl.BlockSpec((bt, DIM), lambda i: (i, 0)),
            pl.BlockSpec((HID, DIM), lambda i: (0, 0)),
            pl.BlockSpec((DIM, HID), lambda i: (0, 0)),
        ],
        out_specs=pl.BlockSpec((bt, DIM), lambda i: (i, 0)),
        out_shape=jax.ShapeDtypeStruct((T, DIM), jnp.float32),
    )(x, su, sd)


def _combine_jnp(shared, out_sorted, row_token, nsteps):
    valid = jnp.arange(SROWS) < nsteps[0] * B_TILE
    rows = jnp.where(valid[:, None], out_sorted, 0.0)
    return shared.at[row_token[:SROWS]].add(rows)


def kernel(x, gate, e_bias, w1, w2, su, sd):
    sel, probs = _routing(x, gate, e_bias)
    row_token, row_prob, counts = _dispatch_jnp(sel, probs)
    step_tile, step_expert, nsteps = _metadata(counts)
    x_sorted = _gather_jnp(x, row_token)
    out_sorted = _gmm(x_sorted, w1, w2, row_prob, step_tile, step_expert,
                      nsteps)
    shared = _shared(x, su, sd)
    return _combine_jnp(shared, out_sorted, row_token, nsteps)
